# Initial kernel scaffold; baseline (speedup 1.0000x reference)
import jax
import jax.numpy as jnp
from jax.experimental import pallas as pl


def kernel(x, edge_index, class_emb, geom_W, geom_b, W1_l, W1_r, b1, W2_l, W2_r, b2, mlp_W1, mlp_b1, mlp_W2, mlp_b2):
    raise NotImplementedError("write your pallas kernel here")


# speedup vs baseline: 4.3609x; 4.3609x over previous
import functools

import jax
import jax.numpy as jnp
from jax import lax
from jax.experimental import pallas as pl
from jax.experimental.pallas import tpu as pltpu
from jax.experimental.pallas import tpu_sc as plsc

N = 50000
E = 800000
HID = 64
NCLS = 32

NC = 2
NS = 16
L = 16

HALF = N // NC
DUMMY = HALF
STRIPE = 1568
HP = NS * STRIPE
SP_C = 3128
NPC = NS * SP_C
C = 128
NCHUNK = E // C
NW = NC * NS

_mesh = plsc.VectorSubcoreMesh(
    core_axis_name="c", subcore_axis_name="s", num_cores=NC, num_subcores=NS)

_sc_params = pltpu.CompilerParams(use_tc_tiling_on_sc=False,
                                  needs_layout_passes=False)

f32 = jnp.float32


def _segsum_body(g_hbm, src_hbm, dst_hbm, z64_hbm,
                 acc_out,
                 acc_sh, sidx, didx, rows, sem):
    c = lax.axis_index("c")
    s = lax.axis_index("s")
    lo = c * HALF
    base_r = s * STRIPE

    pltpu.sync_copy(z64_hbm, acc_sh.at[pl.ds(base_r, STRIPE)])
    plsc.subcore_barrier()

    nch = jnp.where(s < (NCHUNK % NS), NCHUNK // NS + 1, NCHUNK // NS)

    def step(t, carry):
        cb = (s + t * NS) * C
        pltpu.sync_copy(src_hbm.at[pl.ds(cb, C)], sidx)
        pltpu.sync_copy(dst_hbm.at[pl.ds(cb, C)], didx)
        for j in range(C // L):
            d = didx[pl.ds(j * L, L)]
            local = d - lo
            inb = (local >= 0) & (local < HALF)
            didx[pl.ds(j * L, L)] = jnp.where(inb, local, DUMMY)
        pltpu.async_copy(g_hbm.at[sidx], rows, sem).wait()
        pltpu.sync_copy(rows, acc_sh.at[didx], add=True)
        return carry

    lax.fori_loop(0, nch, step, 0)
    plsc.subcore_barrier()

    pltpu.sync_copy(acc_sh.at[pl.ds(base_r, STRIPE)],
                    acc_out.at[c, pl.ds(base_r, STRIPE)])


_segsum_call = pl.kernel(
    _segsum_body,
    out_type=jax.ShapeDtypeStruct((NC, HP, HID), f32),
    mesh=_mesh,
    compiler_params=_sc_params,
    scratch_types=[
        pltpu.VMEM_SHARED((HP, HID), f32),
        pltpu.VMEM((C,), jnp.int32),
        pltpu.VMEM((C,), jnp.int32),
        pltpu.VMEM((C, HID), f32),
        pltpu.SemaphoreType.DMA,
    ],
)


def _count_body(dst_hbm, z16_hbm, ones_hbm,
                cnt_out,
                cnt_sh, didx, ones_v):
    c = lax.axis_index("c")
    s = lax.axis_index("s")
    w = s * NC + c
    base_r = s * SP_C

    pltpu.sync_copy(z16_hbm, cnt_sh.at[pl.ds(base_r, SP_C)])
    pltpu.sync_copy(ones_hbm, ones_v)
    plsc.subcore_barrier()

    nch = jnp.where(w < (NCHUNK % NW), NCHUNK // NW + 1, NCHUNK // NW)

    def step(t, carry):
        cb = (w + t * NW) * C
        pltpu.sync_copy(dst_hbm.at[pl.ds(cb, C)], didx)
        pltpu.sync_copy(ones_v, cnt_sh.at[didx], add=True)
        return carry

    lax.fori_loop(0, nch, step, 0)
    plsc.subcore_barrier()

    pltpu.sync_copy(cnt_sh.at[pl.ds(base_r, SP_C)],
                    cnt_out.at[c, pl.ds(base_r, SP_C)])


_count_call = pl.kernel(
    _count_body,
    out_type=jax.ShapeDtypeStruct((NC, NPC, L), f32),
    mesh=_mesh,
    compiler_params=_sc_params,
    scratch_types=[
        pltpu.VMEM_SHARED((NPC, L), f32),
        pltpu.VMEM((C,), jnp.int32),
        pltpu.VMEM((C, L), f32),
    ],
)


def _edge_mlp_body(a_hbm, b_hbm, src_hbm, dst_hbm, w2_hbm, out_hbm,
                   sidx, didx, arows, brows, w2v, obuf, sem_a, sem_b):
    c = lax.axis_index("c")
    s = lax.axis_index("s")
    w = s * NC + c

    pltpu.sync_copy(w2_hbm, w2v)
    wv = [w2v[pl.ds(k * L, L)] for k in range(HID // L)]
    b2vec = w2v[pl.ds(HID, L)]
    lane = lax.broadcasted_iota(jnp.int32, (L,), 0)

    nw = NC * NS
    nch = jnp.where(w < (NCHUNK % nw), NCHUNK // nw + 1, NCHUNK // nw)

    def step(t, carry):
        cb = (w + t * nw) * C
        pltpu.sync_copy(src_hbm.at[pl.ds(cb, C)], sidx)
        pltpu.sync_copy(dst_hbm.at[pl.ds(cb, C)], didx)
        da = pltpu.async_copy(a_hbm.at[sidx], arows, sem_a)
        db = pltpu.async_copy(b_hbm.at[didx], brows, sem_b)
        da.wait()
        db.wait()

        def group(g, carry2):
            ovec = jnp.zeros((L,), f32)
            for e16 in range(L):
                e = g * L + e16
                acc = b2vec
                for k in range(HID // L):
                    v = arows[e, pl.ds(k * L, L)] + brows[e, pl.ds(k * L, L)]
                    acc = acc + jnp.maximum(v, 0.0) * wv[k]
                ovec = jnp.where(lane == e16, jnp.sum(acc), ovec)
            obuf[pl.ds(g * L, L)] = ovec
            return carry2

        lax.fori_loop(0, C // L, group, 0)
        pltpu.sync_copy(obuf, out_hbm.at[pl.ds(cb, C)])
        return carry

    lax.fori_loop(0, nch, step, 0)


_edge_mlp_call = pl.kernel(
    _edge_mlp_body,
    out_type=jax.ShapeDtypeStruct((E,), f32),
    mesh=_mesh,
    compiler_params=_sc_params,
    scratch_types=[
        pltpu.VMEM((C,), jnp.int32),
        pltpu.VMEM((C,), jnp.int32),
        pltpu.VMEM((C, HID), f32),
        pltpu.VMEM((C, HID), f32),
        pltpu.VMEM((80,), f32),
        pltpu.VMEM((C,), f32),
        pltpu.SemaphoreType.DMA,
        pltpu.SemaphoreType.DMA,
    ],
)


R_ENC = 2000


def _dot(a, b):
    return jnp.dot(a, b, precision=jax.lax.Precision.HIGHEST,
                   preferred_element_type=f32)


def _encode_body(x_ref, tg_ref, gg_ref, cg_ref, tr_ref, gr_ref, cr_ref,
                 g1_ref, r1_ref):
    xb = x_ref[...]
    cls = xb[:, 0:1].astype(jnp.int32)
    onehot = (lax.broadcasted_iota(jnp.int32, (R_ENC, NCLS), 1) == cls)
    onehot = onehot.astype(f32)
    geom = xb[:, 1:5]
    g1_ref[...] = _dot(onehot, tg_ref[...]) + _dot(geom, gg_ref[...]) + cg_ref[...]
    r1_ref[...] = _dot(onehot, tr_ref[...]) + _dot(geom, gr_ref[...]) + cr_ref[...]


def _encode(x, tg, gg, cg, tr, gr, cr):
    full = lambda shape: pl.BlockSpec(shape, lambda i: (0, 0))
    return pl.pallas_call(
        _encode_body,
        grid=(N // R_ENC,),
        in_specs=[
            pl.BlockSpec((R_ENC, 5), lambda i: (i, 0)),
            full((NCLS, HID)), full((4, HID)), full((1, HID)),
            full((NCLS, HID)), full((4, HID)), full((1, HID)),
        ],
        out_specs=[
            pl.BlockSpec((R_ENC, HID), lambda i: (i, 0)),
            pl.BlockSpec((R_ENC, HID), lambda i: (i, 0)),
        ],
        out_shape=[
            jax.ShapeDtypeStruct((N, HID), f32),
            jax.ShapeDtypeStruct((N, HID), f32),
        ],
    )(x, tg, gg, cg, tr, gr, cr)


R_FIN = 5000
_HB = HALF // R_FIN


def _finish1_body(acc_ref, cnt_ref, r1_ref, wl_ref, wr_ref, b_ref,
                  g2_ref, r2_ref):
    cnt = (cnt_ref[0] + cnt_ref[1])[:, 0:1]
    inv = 1.0 / jnp.maximum(cnt, 1.0)
    h1 = jnp.maximum(acc_ref[0] * inv + r1_ref[...], 0.0)
    g2_ref[...] = _dot(h1, wl_ref[...])
    r2_ref[...] = _dot(h1, wr_ref[...]) + b_ref[...]


def _finish2_body(acc_ref, cnt_ref, r2_ref, wa_ref, wb_ref, b_ref,
                  a_ref, b_out_ref):
    cnt = (cnt_ref[0] + cnt_ref[1])[:, 0:1]
    inv = 1.0 / jnp.maximum(cnt, 1.0)
    h2 = acc_ref[0] * inv + r2_ref[...]
    a_ref[...] = _dot(h2, wa_ref[...]) + b_ref[...]
    b_out_ref[...] = _dot(h2, wb_ref[...])


def _finish(body, acc, cnt, r, wl, wr, b):
    full = lambda shape: pl.BlockSpec(shape, lambda h, i: (0, 0))
    return pl.pallas_call(
        body,
        grid=(NC, _HB),
        in_specs=[
            pl.BlockSpec((1, R_FIN, HID), lambda h, i: (h, i, 0)),
            pl.BlockSpec((NC, R_FIN, L), lambda h, i: (0, h * _HB + i, 0)),
            pl.BlockSpec((R_FIN, HID), lambda h, i: (h * _HB + i, 0)),
            full((HID, HID)), full((HID, HID)), full((1, HID)),
        ],
        out_specs=[
            pl.BlockSpec((R_FIN, HID), lambda h, i: (h * _HB + i, 0)),
            pl.BlockSpec((R_FIN, HID), lambda h, i: (h * _HB + i, 0)),
        ],
        out_shape=[
            jax.ShapeDtypeStruct((N, HID), f32),
            jax.ShapeDtypeStruct((N, HID), f32),
        ],
    )(acc, cnt, r, wl, wr, b)


def kernel(x, edge_index, class_emb, geom_W, geom_b, W1_l, W1_r, b1,
           W2_l, W2_r, b2, mlp_W1, mlp_b1, mlp_W2, mlp_b2):
    src = edge_index[0]
    dst = edge_index[1]

    W1l_a, W1l_b = W1_l[:HID], W1_l[HID:]
    W1r_a, W1r_b = W1_r[:HID], W1_r[HID:]
    tg = class_emb @ W1l_a
    gg = geom_W @ W1l_b
    cg = (geom_b @ W1l_b)[None]
    tr = class_emb @ W1r_a
    gr = geom_W @ W1r_b
    cr = (geom_b @ W1r_b + b1)[None]
    wa = mlp_W1[:HID]
    wb = mlp_W1[HID:]
    w2pad = jnp.zeros((80,), f32).at[:HID].set(mlp_W2[:, 0]).at[HID].set(mlp_b2[0])

    z64 = jnp.zeros((STRIPE, HID), f32)
    z16 = jnp.zeros((SP_C, L), f32)
    ones = jnp.ones((C, L), f32)

    g1, r1 = _encode(x, tg, gg, cg, tr, gr, cr)
    cnt = _count_call(dst, z16, ones)
    acc1 = _segsum_call(g1, src, dst, z64)
    g2, r2 = _finish(_finish1_body, acc1, cnt, r1, W2_l, W2_r, b2[None])
    acc2 = _segsum_call(g2, src, dst, z64)
    a_tab, b_tab = _finish(_finish2_body, acc2, cnt, r2, wa, wb, mlp_b1[None])
    return _edge_mlp_call(a_tab, b_tab, src, dst, w2pad)


# speedup vs baseline: 6.2310x; 1.4288x over previous
import functools

import jax
import jax.numpy as jnp
from jax import lax
from jax.experimental import pallas as pl
from jax.experimental.pallas import tpu as pltpu
from jax.experimental.pallas import tpu_sc as plsc

N = 50000
E = 800000
HID = 64
NCLS = 32

NC = 2
NS = 16
L = 16

HALF = N // NC
DUMMY = HALF
STRIPE = 1568
HP = NS * STRIPE
SP_C = 3128
NPC = NS * SP_C
C = 400
NCHUNK = E // C
NW = NC * NS

_mesh = plsc.VectorSubcoreMesh(
    core_axis_name="c", subcore_axis_name="s", num_cores=NC, num_subcores=NS)

_sc_params = pltpu.CompilerParams(use_tc_tiling_on_sc=False,
                                  needs_layout_passes=False)

f32 = jnp.float32


def _segsum_body(g_hbm, src_hbm, dst_hbm, z64_hbm,
                 acc_out,
                 acc_sh, sidx, didx, rows, sem):
    c = lax.axis_index("c")
    s = lax.axis_index("s")
    lo = c * HALF
    base_r = s * STRIPE

    pltpu.sync_copy(z64_hbm, acc_sh.at[pl.ds(base_r, STRIPE)])
    plsc.subcore_barrier()

    nch = jnp.where(s < (NCHUNK % NS), NCHUNK // NS + 1, NCHUNK // NS)

    def step(t, carry):
        cb = (s + t * NS) * C
        pltpu.sync_copy(src_hbm.at[pl.ds(cb, C)], sidx)
        pltpu.sync_copy(dst_hbm.at[pl.ds(cb, C)], didx)
        for j in range(C // L):
            d = didx[pl.ds(j * L, L)]
            local = d - lo
            inb = (local >= 0) & (local < HALF)
            didx[pl.ds(j * L, L)] = jnp.where(inb, local, DUMMY)
        pltpu.async_copy(g_hbm.at[sidx], rows, sem).wait()
        pltpu.sync_copy(rows, acc_sh.at[didx], add=True)
        return carry

    lax.fori_loop(0, nch, step, 0)
    plsc.subcore_barrier()

    pltpu.sync_copy(acc_sh.at[pl.ds(base_r, STRIPE)],
                    acc_out.at[c, pl.ds(base_r, STRIPE)])


_segsum_call = pl.kernel(
    _segsum_body,
    out_type=jax.ShapeDtypeStruct((NC, HP, HID), f32),
    mesh=_mesh,
    compiler_params=_sc_params,
    scratch_types=[
        pltpu.VMEM_SHARED((HP, HID), f32),
        pltpu.VMEM((C,), jnp.int32),
        pltpu.VMEM((C,), jnp.int32),
        pltpu.VMEM((C, HID), f32),
        pltpu.SemaphoreType.DMA,
    ],
)


def _count_body(dst_hbm, z16_hbm, ones_hbm,
                cnt_out,
                cnt_sh, didx, ones_v):
    c = lax.axis_index("c")
    s = lax.axis_index("s")
    w = s * NC + c
    base_r = s * SP_C

    pltpu.sync_copy(z16_hbm, cnt_sh.at[pl.ds(base_r, SP_C)])
    pltpu.sync_copy(ones_hbm, ones_v)
    plsc.subcore_barrier()

    nch = jnp.where(w < (NCHUNK % NW), NCHUNK // NW + 1, NCHUNK // NW)

    def step(t, carry):
        cb = (w + t * NW) * C
        pltpu.sync_copy(dst_hbm.at[pl.ds(cb, C)], didx)
        pltpu.sync_copy(ones_v, cnt_sh.at[didx], add=True)
        return carry

    lax.fori_loop(0, nch, step, 0)
    plsc.subcore_barrier()

    pltpu.sync_copy(cnt_sh.at[pl.ds(base_r, SP_C)],
                    cnt_out.at[c, pl.ds(base_r, SP_C)])


_count_call = pl.kernel(
    _count_body,
    out_type=jax.ShapeDtypeStruct((NC, NPC, L), f32),
    mesh=_mesh,
    compiler_params=_sc_params,
    scratch_types=[
        pltpu.VMEM_SHARED((NPC, L), f32),
        pltpu.VMEM((C,), jnp.int32),
        pltpu.VMEM((C, L), f32),
    ],
)


def _edge_mlp_body(a_hbm, b_hbm, src_hbm, dst_hbm, w2_hbm, out_hbm,
                   sidx, didx, arows, brows, w2v, obuf, sem_a, sem_b):
    c = lax.axis_index("c")
    s = lax.axis_index("s")
    w = s * NC + c

    pltpu.sync_copy(w2_hbm, w2v)
    wv = [w2v[pl.ds(k * L, L)] for k in range(HID // L)]
    b2vec = w2v[pl.ds(HID, L)]
    lane = lax.broadcasted_iota(jnp.int32, (L,), 0)

    nw = NC * NS
    nch = jnp.where(w < (NCHUNK % nw), NCHUNK // nw + 1, NCHUNK // nw)

    def step(t, carry):
        cb = (w + t * nw) * C
        pltpu.sync_copy(src_hbm.at[pl.ds(cb, C)], sidx)
        pltpu.sync_copy(dst_hbm.at[pl.ds(cb, C)], didx)
        da = pltpu.async_copy(a_hbm.at[sidx], arows, sem_a)
        db = pltpu.async_copy(b_hbm.at[didx], brows, sem_b)
        da.wait()
        db.wait()

        def group(g, carry2):
            ovec = jnp.zeros((L,), f32)
            for e16 in range(L):
                e = g * L + e16
                acc = b2vec
                for k in range(HID // L):
                    v = arows[e, pl.ds(k * L, L)] + brows[e, pl.ds(k * L, L)]
                    acc = acc + jnp.maximum(v, 0.0) * wv[k]
                ovec = jnp.where(lane == e16, jnp.sum(acc), ovec)
            obuf[pl.ds(g * L, L)] = ovec
            return carry2

        lax.fori_loop(0, C // L, group, 0)
        pltpu.sync_copy(obuf, out_hbm.at[pl.ds(cb, C)])
        return carry

    lax.fori_loop(0, nch, step, 0)


_edge_mlp_call = pl.kernel(
    _edge_mlp_body,
    out_type=jax.ShapeDtypeStruct((E,), f32),
    mesh=_mesh,
    compiler_params=_sc_params,
    scratch_types=[
        pltpu.VMEM((C,), jnp.int32),
        pltpu.VMEM((C,), jnp.int32),
        pltpu.VMEM((C, HID), f32),
        pltpu.VMEM((C, HID), f32),
        pltpu.VMEM((80,), f32),
        pltpu.VMEM((C,), f32),
        pltpu.SemaphoreType.DMA,
        pltpu.SemaphoreType.DMA,
    ],
)


R_ENC = 2000


def _dot(a, b):
    return jnp.dot(a, b, precision=jax.lax.Precision.HIGHEST,
                   preferred_element_type=f32)


def _encode_body(x_ref, tg_ref, gg_ref, cg_ref, tr_ref, gr_ref, cr_ref,
                 g1_ref, r1_ref):
    xb = x_ref[...]
    cls = xb[:, 0:1].astype(jnp.int32)
    onehot = (lax.broadcasted_iota(jnp.int32, (R_ENC, NCLS), 1) == cls)
    onehot = onehot.astype(f32)
    geom = xb[:, 1:5]
    g1_ref[...] = _dot(onehot, tg_ref[...]) + _dot(geom, gg_ref[...]) + cg_ref[...]
    r1_ref[...] = _dot(onehot, tr_ref[...]) + _dot(geom, gr_ref[...]) + cr_ref[...]


def _encode(x, tg, gg, cg, tr, gr, cr):
    full = lambda shape: pl.BlockSpec(shape, lambda i: (0, 0))
    return pl.pallas_call(
        _encode_body,
        grid=(N // R_ENC,),
        in_specs=[
            pl.BlockSpec((R_ENC, 5), lambda i: (i, 0)),
            full((NCLS, HID)), full((4, HID)), full((1, HID)),
            full((NCLS, HID)), full((4, HID)), full((1, HID)),
        ],
        out_specs=[
            pl.BlockSpec((R_ENC, HID), lambda i: (i, 0)),
            pl.BlockSpec((R_ENC, HID), lambda i: (i, 0)),
        ],
        out_shape=[
            jax.ShapeDtypeStruct((N, HID), f32),
            jax.ShapeDtypeStruct((N, HID), f32),
        ],
    )(x, tg, gg, cg, tr, gr, cr)


R_FIN = 5000
_HB = HALF // R_FIN


def _finish1_body(acc_ref, cnt_ref, r1_ref, wl_ref, wr_ref, b_ref,
                  g2_ref, r2_ref):
    cnt = (cnt_ref[0] + cnt_ref[1])[:, 0:1]
    inv = 1.0 / jnp.maximum(cnt, 1.0)
    h1 = jnp.maximum(acc_ref[0] * inv + r1_ref[...], 0.0)
    g2_ref[...] = _dot(h1, wl_ref[...])
    r2_ref[...] = _dot(h1, wr_ref[...]) + b_ref[...]


def _finish2_body(acc_ref, cnt_ref, r2_ref, wa_ref, wb_ref, b_ref,
                  a_ref, b_out_ref):
    cnt = (cnt_ref[0] + cnt_ref[1])[:, 0:1]
    inv = 1.0 / jnp.maximum(cnt, 1.0)
    h2 = acc_ref[0] * inv + r2_ref[...]
    a_ref[...] = _dot(h2, wa_ref[...]) + b_ref[...]
    b_out_ref[...] = _dot(h2, wb_ref[...])


def _finish(body, acc, cnt, r, wl, wr, b):
    full = lambda shape: pl.BlockSpec(shape, lambda h, i: (0, 0))
    return pl.pallas_call(
        body,
        grid=(NC, _HB),
        in_specs=[
            pl.BlockSpec((1, R_FIN, HID), lambda h, i: (h, i, 0)),
            pl.BlockSpec((NC, R_FIN, L), lambda h, i: (0, h * _HB + i, 0)),
            pl.BlockSpec((R_FIN, HID), lambda h, i: (h * _HB + i, 0)),
            full((HID, HID)), full((HID, HID)), full((1, HID)),
        ],
        out_specs=[
            pl.BlockSpec((R_FIN, HID), lambda h, i: (h * _HB + i, 0)),
            pl.BlockSpec((R_FIN, HID), lambda h, i: (h * _HB + i, 0)),
        ],
        out_shape=[
            jax.ShapeDtypeStruct((N, HID), f32),
            jax.ShapeDtypeStruct((N, HID), f32),
        ],
    )(acc, cnt, r, wl, wr, b)


def kernel(x, edge_index, class_emb, geom_W, geom_b, W1_l, W1_r, b1,
           W2_l, W2_r, b2, mlp_W1, mlp_b1, mlp_W2, mlp_b2):
    src = edge_index[0]
    dst = edge_index[1]

    W1l_a, W1l_b = W1_l[:HID], W1_l[HID:]
    W1r_a, W1r_b = W1_r[:HID], W1_r[HID:]
    tg = class_emb @ W1l_a
    gg = geom_W @ W1l_b
    cg = (geom_b @ W1l_b)[None]
    tr = class_emb @ W1r_a
    gr = geom_W @ W1r_b
    cr = (geom_b @ W1r_b + b1)[None]
    wa = mlp_W1[:HID]
    wb = mlp_W1[HID:]
    w2pad = jnp.zeros((80,), f32).at[:HID].set(mlp_W2[:, 0]).at[HID].set(mlp_b2[0])

    z64 = jnp.zeros((STRIPE, HID), f32)
    z16 = jnp.zeros((SP_C, L), f32)
    ones = jnp.ones((C, L), f32)

    g1, r1 = _encode(x, tg, gg, cg, tr, gr, cr)
    cnt = _count_call(dst, z16, ones)
    acc1 = _segsum_call(g1, src, dst, z64)
    g2, r2 = _finish(_finish1_body, acc1, cnt, r1, W2_l, W2_r, b2[None])
    acc2 = _segsum_call(g2, src, dst, z64)
    a_tab, b_tab = _finish(_finish2_body, acc2, cnt, r2, wa, wb, mlp_b1[None])
    return _edge_mlp_call(a_tab, b_tab, src, dst, w2pad)


# speedup vs baseline: 6.6442x; 1.0663x over previous
import functools

import jax
import jax.numpy as jnp
from jax import lax
from jax.experimental import pallas as pl
from jax.experimental.pallas import tpu as pltpu
from jax.experimental.pallas import tpu_sc as plsc

N = 50000
E = 800000
HID = 64
NCLS = 32

NC = 2
NS = 16
L = 16

HALF = N // NC
DUMMY = HALF
STRIPE = 1568
HP = NS * STRIPE
SP_C = 3128
NPC = NS * SP_C
CS = 160
NCHS = E // CS
CE = 320
NCHE = E // CE
NW = NC * NS

_mesh = plsc.VectorSubcoreMesh(
    core_axis_name="c", subcore_axis_name="s", num_cores=NC, num_subcores=NS)

_sc_params = pltpu.CompilerParams(use_tc_tiling_on_sc=False,
                                  needs_layout_passes=False)

f32 = jnp.float32


def _segsum_body(g_hbm, src_hbm, dst_hbm, z64_hbm,
                 acc_out,
                 acc_sh, sidx, didx, rows, sem0, sem1):
    c = lax.axis_index("c")
    s = lax.axis_index("s")
    lo = c * HALF
    base_r = s * STRIPE
    sems = (sem0, sem1)

    pltpu.sync_copy(z64_hbm, acc_sh.at[pl.ds(base_r, STRIPE)])
    plsc.subcore_barrier()

    nch = jnp.where(s < (NCHS % NS), NCHS // NS + 1, NCHS // NS)

    def fire(k, b):
        cb = (s + k * NS) * CS
        pltpu.sync_copy(src_hbm.at[pl.ds(cb, CS)], sidx.at[b])
        pltpu.sync_copy(dst_hbm.at[pl.ds(cb, CS)], didx.at[b])
        for j in range(CS // L):
            d = didx[b, pl.ds(j * L, L)]
            local = d - lo
            inb = (local >= 0) & (local < HALF)
            didx[b, pl.ds(j * L, L)] = jnp.where(inb, local, DUMMY)
        pltpu.async_copy(g_hbm.at[sidx.at[b]], rows.at[b], sems[b])

    def drain(b):
        pltpu.make_async_copy(g_hbm.at[sidx.at[b]], rows.at[b], sems[b]).wait()
        pltpu.sync_copy(rows.at[b], acc_sh.at[didx.at[b]], add=True)

    fire(0, 0)
    n_t2 = (NCHS // NS + 2) // 2

    def step(t2, carry):
        for b in (0, 1):
            k = t2 * 2 + b
            pl.when(k + 1 < nch)(lambda: fire(k + 1, 1 - b))
            pl.when(k < nch)(lambda: drain(b))
        return carry

    lax.fori_loop(0, n_t2, step, 0)
    plsc.subcore_barrier()

    pltpu.sync_copy(acc_sh.at[pl.ds(base_r, STRIPE)],
                    acc_out.at[c, pl.ds(base_r, STRIPE)])


_segsum_call = pl.kernel(
    _segsum_body,
    out_type=jax.ShapeDtypeStruct((NC, HP, HID), f32),
    mesh=_mesh,
    compiler_params=_sc_params,
    scratch_types=[
        pltpu.VMEM_SHARED((HP, HID), f32),
        pltpu.VMEM((2, CS), jnp.int32),
        pltpu.VMEM((2, CS), jnp.int32),
        pltpu.VMEM((2, CS, HID), f32),
        pltpu.SemaphoreType.DMA,
        pltpu.SemaphoreType.DMA,
    ],
)


def _count_body(dst_hbm, z16_hbm, ones_hbm,
                cnt_out,
                cnt_sh, didx, ones_v):
    c = lax.axis_index("c")
    s = lax.axis_index("s")
    w = s * NC + c
    base_r = s * SP_C

    pltpu.sync_copy(z16_hbm, cnt_sh.at[pl.ds(base_r, SP_C)])
    pltpu.sync_copy(ones_hbm, ones_v)
    plsc.subcore_barrier()

    nch = jnp.where(w < (NCHS % NW), NCHS // NW + 1, NCHS // NW)

    def step(t, carry):
        cb = (w + t * NW) * CS
        pltpu.sync_copy(dst_hbm.at[pl.ds(cb, CS)], didx)
        pltpu.sync_copy(ones_v, cnt_sh.at[didx], add=True)
        return carry

    lax.fori_loop(0, nch, step, 0)
    plsc.subcore_barrier()

    pltpu.sync_copy(cnt_sh.at[pl.ds(base_r, SP_C)],
                    cnt_out.at[c, pl.ds(base_r, SP_C)])


_count_call = pl.kernel(
    _count_body,
    out_type=jax.ShapeDtypeStruct((NC, NPC, L), f32),
    mesh=_mesh,
    compiler_params=_sc_params,
    scratch_types=[
        pltpu.VMEM_SHARED((NPC, L), f32),
        pltpu.VMEM((CS,), jnp.int32),
        pltpu.VMEM((CS, L), f32),
    ],
)


def _edge_mlp_body(a_hbm, b_hbm, src_hbm, dst_hbm, w2_hbm, out_hbm,
                   sidx, didx, arows, brows, w2v, obuf,
                   sem_a0, sem_a1, sem_b0, sem_b1):
    c = lax.axis_index("c")
    s = lax.axis_index("s")
    w = s * NC + c
    sems_a = (sem_a0, sem_a1)
    sems_b = (sem_b0, sem_b1)

    pltpu.sync_copy(w2_hbm, w2v)
    wv = [w2v[pl.ds(k * L, L)] for k in range(HID // L)]
    b2vec = w2v[pl.ds(HID, L)]
    lane = lax.broadcasted_iota(jnp.int32, (L,), 0)

    nch = jnp.where(w < (NCHE % NW), NCHE // NW + 1, NCHE // NW)

    def fire(k, b):
        cb = (w + k * NW) * CE
        pltpu.sync_copy(src_hbm.at[pl.ds(cb, CE)], sidx.at[b])
        pltpu.sync_copy(dst_hbm.at[pl.ds(cb, CE)], didx.at[b])
        pltpu.async_copy(a_hbm.at[sidx.at[b]], arows.at[b], sems_a[b])
        pltpu.async_copy(b_hbm.at[didx.at[b]], brows.at[b], sems_b[b])

    def drain(k, b):
        cb = (w + k * NW) * CE
        pltpu.make_async_copy(a_hbm.at[sidx.at[b]], arows.at[b], sems_a[b]).wait()
        pltpu.make_async_copy(b_hbm.at[didx.at[b]], brows.at[b], sems_b[b]).wait()

        def group(g, carry2):
            ovec = jnp.zeros((L,), f32)
            for e16 in range(L):
                e = g * L + e16
                acc = b2vec
                for kk in range(HID // L):
                    v = arows[b, e, pl.ds(kk * L, L)] + brows[b, e, pl.ds(kk * L, L)]
                    acc = acc + jnp.maximum(v, 0.0) * wv[kk]
                ovec = jnp.where(lane == e16, jnp.sum(acc), ovec)
            obuf[pl.ds(g * L, L)] = ovec
            return carry2

        lax.fori_loop(0, CE // L, group, 0)
        pltpu.sync_copy(obuf, out_hbm.at[pl.ds(cb, CE)])

    fire(0, 0)
    n_t2 = (NCHE // NW + 2) // 2

    def step(t2, carry):
        for b in (0, 1):
            k = t2 * 2 + b
            pl.when(k + 1 < nch)(lambda: fire(k + 1, 1 - b))
            pl.when(k < nch)(lambda: drain(k, b))
        return carry

    lax.fori_loop(0, n_t2, step, 0)


_edge_mlp_call = pl.kernel(
    _edge_mlp_body,
    out_type=jax.ShapeDtypeStruct((E,), f32),
    mesh=_mesh,
    compiler_params=_sc_params,
    scratch_types=[
        pltpu.VMEM((2, CE), jnp.int32),
        pltpu.VMEM((2, CE), jnp.int32),
        pltpu.VMEM((2, CE, HID), f32),
        pltpu.VMEM((2, CE, HID), f32),
        pltpu.VMEM((80,), f32),
        pltpu.VMEM((CE,), f32),
        pltpu.SemaphoreType.DMA,
        pltpu.SemaphoreType.DMA,
        pltpu.SemaphoreType.DMA,
        pltpu.SemaphoreType.DMA,
    ],
)


R_ENC = 2000


def _dot(a, b):
    return jnp.dot(a, b, precision=jax.lax.Precision.HIGHEST,
                   preferred_element_type=f32)


def _encode_body(x_ref, tg_ref, gg_ref, cg_ref, tr_ref, gr_ref, cr_ref,
                 g1_ref, r1_ref):
    xb = x_ref[...]
    cls = xb[:, 0:1].astype(jnp.int32)
    onehot = (lax.broadcasted_iota(jnp.int32, (R_ENC, NCLS), 1) == cls)
    onehot = onehot.astype(f32)
    geom = xb[:, 1:5]
    g1_ref[...] = _dot(onehot, tg_ref[...]) + _dot(geom, gg_ref[...]) + cg_ref[...]
    r1_ref[...] = _dot(onehot, tr_ref[...]) + _dot(geom, gr_ref[...]) + cr_ref[...]


def _encode(x, tg, gg, cg, tr, gr, cr):
    full = lambda shape: pl.BlockSpec(shape, lambda i: (0, 0))
    return pl.pallas_call(
        _encode_body,
        grid=(N // R_ENC,),
        in_specs=[
            pl.BlockSpec((R_ENC, 5), lambda i: (i, 0)),
            full((NCLS, HID)), full((4, HID)), full((1, HID)),
            full((NCLS, HID)), full((4, HID)), full((1, HID)),
        ],
        out_specs=[
            pl.BlockSpec((R_ENC, HID), lambda i: (i, 0)),
            pl.BlockSpec((R_ENC, HID), lambda i: (i, 0)),
        ],
        out_shape=[
            jax.ShapeDtypeStruct((N, HID), f32),
            jax.ShapeDtypeStruct((N, HID), f32),
        ],
    )(x, tg, gg, cg, tr, gr, cr)


R_FIN = 5000
_HB = HALF // R_FIN


def _finish1_body(acc_ref, cnt_ref, r1_ref, wl_ref, wr_ref, b_ref,
                  g2_ref, r2_ref):
    cnt = (cnt_ref[0] + cnt_ref[1])[:, 0:1]
    inv = 1.0 / jnp.maximum(cnt, 1.0)
    h1 = jnp.maximum(acc_ref[0] * inv + r1_ref[...], 0.0)
    g2_ref[...] = _dot(h1, wl_ref[...])
    r2_ref[...] = _dot(h1, wr_ref[...]) + b_ref[...]


def _finish2_body(acc_ref, cnt_ref, r2_ref, wa_ref, wb_ref, b_ref,
                  a_ref, b_out_ref):
    cnt = (cnt_ref[0] + cnt_ref[1])[:, 0:1]
    inv = 1.0 / jnp.maximum(cnt, 1.0)
    h2 = acc_ref[0] * inv + r2_ref[...]
    a_ref[...] = _dot(h2, wa_ref[...]) + b_ref[...]
    b_out_ref[...] = _dot(h2, wb_ref[...])


def _finish(body, acc, cnt, r, wl, wr, b):
    full = lambda shape: pl.BlockSpec(shape, lambda h, i: (0, 0))
    return pl.pallas_call(
        body,
        grid=(NC, _HB),
        in_specs=[
            pl.BlockSpec((1, R_FIN, HID), lambda h, i: (h, i, 0)),
            pl.BlockSpec((NC, R_FIN, L), lambda h, i: (0, h * _HB + i, 0)),
            pl.BlockSpec((R_FIN, HID), lambda h, i: (h * _HB + i, 0)),
            full((HID, HID)), full((HID, HID)), full((1, HID)),
        ],
        out_specs=[
            pl.BlockSpec((R_FIN, HID), lambda h, i: (h * _HB + i, 0)),
            pl.BlockSpec((R_FIN, HID), lambda h, i: (h * _HB + i, 0)),
        ],
        out_shape=[
            jax.ShapeDtypeStruct((N, HID), f32),
            jax.ShapeDtypeStruct((N, HID), f32),
        ],
    )(acc, cnt, r, wl, wr, b)


def kernel(x, edge_index, class_emb, geom_W, geom_b, W1_l, W1_r, b1,
           W2_l, W2_r, b2, mlp_W1, mlp_b1, mlp_W2, mlp_b2):
    src = edge_index[0]
    dst = edge_index[1]

    W1l_a, W1l_b = W1_l[:HID], W1_l[HID:]
    W1r_a, W1r_b = W1_r[:HID], W1_r[HID:]
    tg = class_emb @ W1l_a
    gg = geom_W @ W1l_b
    cg = (geom_b @ W1l_b)[None]
    tr = class_emb @ W1r_a
    gr = geom_W @ W1r_b
    cr = (geom_b @ W1r_b + b1)[None]
    wa = mlp_W1[:HID]
    wb = mlp_W1[HID:]
    w2pad = jnp.zeros((80,), f32).at[:HID].set(mlp_W2[:, 0]).at[HID].set(mlp_b2[0])

    z64 = jnp.zeros((STRIPE, HID), f32)
    z16 = jnp.zeros((SP_C, L), f32)
    ones = jnp.ones((CS, L), f32)

    g1, r1 = _encode(x, tg, gg, cg, tr, gr, cr)
    cnt = _count_call(dst, z16, ones)
    acc1 = _segsum_call(g1, src, dst, z64)
    g2, r2 = _finish(_finish1_body, acc1, cnt, r1, W2_l, W2_r, b2[None])
    acc2 = _segsum_call(g2, src, dst, z64)
    a_tab, b_tab = _finish(_finish2_body, acc2, cnt, r2, wa, wb, mlp_b1[None])
    return _edge_mlp_call(a_tab, b_tab, src, dst, w2pad)


# speedup vs baseline: 9.4073x; 1.4159x over previous
import functools

import jax
import jax.numpy as jnp
from jax import lax
from jax.experimental import pallas as pl
from jax.experimental.pallas import tpu as pltpu
from jax.experimental.pallas import tpu_sc as plsc

N = 50000
E = 800000
HID = 64
NCLS = 32

NC = 2
NS = 16
L = 16

HALF = N // NC
DUMMY = HALF
STRIPE = 1568
HP = NS * STRIPE
SP_C = 3128
NPC = NS * SP_C
CS = 160
NCHS = E // CS
CE = 320
NCHE = E // CE
NW = NC * NS
EPT = E // NS
CP = 2000
ECAP = E + NS * CS

_mesh = plsc.VectorSubcoreMesh(
    core_axis_name="c", subcore_axis_name="s", num_cores=NC, num_subcores=NS)

_sc_params = pltpu.CompilerParams(use_tc_tiling_on_sc=False,
                                  needs_layout_passes=False)

f32 = jnp.float32


def _partition_body(src_hbm, dst_hbm,
                    psrc_out, pdst_out, meta_out,
                    ssrc, sdst, sbuf, dbuf, mbuf, cntr):
    c = lax.axis_index("c")
    s = lax.axis_index("s")
    lo = c * HALF
    lane = lax.broadcasted_iota(jnp.int32, (L,), 0)

    @pl.when(s == 0)
    def _():
        cntr[0] = 0
    plsc.subcore_barrier()

    base_e = s * EPT

    def scan_chunk(q, n):
        pltpu.sync_copy(src_hbm.at[pl.ds(base_e + q * CP, CP)], sbuf)
        pltpu.sync_copy(dst_hbm.at[pl.ds(base_e + q * CP, CP)], dbuf)

        def group(j, n2):
            sv = sbuf[pl.ds(j * L, L)]
            dv = dbuf[pl.ds(j * L, L)]
            dloc = dv - lo
            m = (dloc >= 0) & (dloc < HALF)
            plsc.store_compressed(ssrc.at[pl.ds(n2, L)], sv, mask=m)
            plsc.store_compressed(sdst.at[pl.ds(n2, L)], dloc, mask=m)
            return n2 + jnp.sum(m.astype(jnp.int32))

        return lax.fori_loop(0, CP // L, group, n)

    n_loc = lax.fori_loop(0, EPT // CP, scan_chunk, jnp.int32(0))

    zl = jnp.zeros((L,), jnp.int32)
    dl = jnp.full((L,), DUMMY, jnp.int32)
    for j in range(CS // L):
        ssrc[pl.ds(n_loc + j * L, L)] = zl
        sdst[pl.ds(n_loc + j * L, L)] = dl
    nch_loc = (n_loc + CS - 1) // CS
    base = plsc.fetch_and_add(cntr.at[0], nch_loc * CS, subcore_id=0)

    def flush(i, carry):
        ob = pl.multiple_of(base + i * CS, CS)
        pltpu.sync_copy(ssrc.at[pl.ds(i * CS, CS)],
                        psrc_out.at[c, pl.ds(ob, CS)])
        pltpu.sync_copy(sdst.at[pl.ds(i * CS, CS)],
                        pdst_out.at[c, pl.ds(ob, CS)])
        return carry

    lax.fori_loop(0, nch_loc, flush, 0)
    plsc.subcore_barrier()

    @pl.when(s == 0)
    def _():
        total = plsc.fetch_and_add(cntr.at[0], 0, subcore_id=0)
        mbuf[...] = jnp.where(lane == 0, total, 0)
        pltpu.sync_copy(mbuf, meta_out.at[c])


_partition_call = pl.kernel(
    _partition_body,
    out_type=(jax.ShapeDtypeStruct((NC, ECAP), jnp.int32),
              jax.ShapeDtypeStruct((NC, ECAP), jnp.int32),
              jax.ShapeDtypeStruct((NC, L), jnp.int32)),
    mesh=_mesh,
    compiler_params=_sc_params,
    scratch_types=[
        pltpu.VMEM((EPT + CS,), jnp.int32),
        pltpu.VMEM((EPT + CS,), jnp.int32),
        pltpu.VMEM((CP,), jnp.int32),
        pltpu.VMEM((CP,), jnp.int32),
        pltpu.VMEM((L,), jnp.int32),
        pltpu.SMEM((1,), jnp.int32),
    ],
)


def _segsum_body(g_hbm, psrc_hbm, pdst_hbm, meta_hbm, z64_hbm,
                 acc_out,
                 acc_sh, sidx, didx, rows, mbuf, sem0, sem1):
    c = lax.axis_index("c")
    s = lax.axis_index("s")
    base_r = s * STRIPE
    sems = (sem0, sem1)

    pltpu.sync_copy(z64_hbm, acc_sh.at[pl.ds(base_r, STRIPE)])
    plsc.subcore_barrier()

    pltpu.sync_copy(meta_hbm.at[c], mbuf)
    total = mbuf[...][0]
    nchunks = total // CS
    nch = (nchunks - s + NS - 1) // NS

    def fire(k, b):
        cb = pl.multiple_of((s + k * NS) * CS, CS)
        pltpu.sync_copy(psrc_hbm.at[c, pl.ds(cb, CS)], sidx.at[b])
        pltpu.sync_copy(pdst_hbm.at[c, pl.ds(cb, CS)], didx.at[b])
        pltpu.async_copy(g_hbm.at[sidx.at[b]], rows.at[b], sems[b])

    def drain(b):
        pltpu.make_async_copy(g_hbm.at[sidx.at[b]], rows.at[b], sems[b]).wait()
        pltpu.sync_copy(rows.at[b], acc_sh.at[didx.at[b]], add=True)

    pl.when(nch > 0)(lambda: fire(0, 0))
    n_t2 = 158

    def step(t2, carry):
        for b in (0, 1):
            k = t2 * 2 + b
            pl.when(k + 1 < nch)(lambda: fire(k + 1, 1 - b))
            pl.when(k < nch)(lambda: drain(b))
        return carry

    lax.fori_loop(0, n_t2, step, 0)
    plsc.subcore_barrier()

    pltpu.sync_copy(acc_sh.at[pl.ds(base_r, STRIPE)],
                    acc_out.at[c, pl.ds(base_r, STRIPE)])


_segsum_call = pl.kernel(
    _segsum_body,
    out_type=jax.ShapeDtypeStruct((NC, HP, HID), f32),
    mesh=_mesh,
    compiler_params=_sc_params,
    scratch_types=[
        pltpu.VMEM_SHARED((HP, HID), f32),
        pltpu.VMEM((2, CS), jnp.int32),
        pltpu.VMEM((2, CS), jnp.int32),
        pltpu.VMEM((2, CS, HID), f32),
        pltpu.VMEM((L,), jnp.int32),
        pltpu.SemaphoreType.DMA,
        pltpu.SemaphoreType.DMA,
    ],
)


def _count_body(dst_hbm, z16_hbm, ones_hbm,
                cnt_out,
                cnt_sh, didx, ones_v):
    c = lax.axis_index("c")
    s = lax.axis_index("s")
    w = s * NC + c
    base_r = s * SP_C

    pltpu.sync_copy(z16_hbm, cnt_sh.at[pl.ds(base_r, SP_C)])
    pltpu.sync_copy(ones_hbm, ones_v)
    plsc.subcore_barrier()

    nch = jnp.where(w < (NCHS % NW), NCHS // NW + 1, NCHS // NW)

    def step(t, carry):
        cb = (w + t * NW) * CS
        pltpu.sync_copy(dst_hbm.at[pl.ds(cb, CS)], didx)
        pltpu.sync_copy(ones_v, cnt_sh.at[didx], add=True)
        return carry

    lax.fori_loop(0, nch, step, 0)
    plsc.subcore_barrier()

    pltpu.sync_copy(cnt_sh.at[pl.ds(base_r, SP_C)],
                    cnt_out.at[c, pl.ds(base_r, SP_C)])


_count_call = pl.kernel(
    _count_body,
    out_type=jax.ShapeDtypeStruct((NC, NPC, L), f32),
    mesh=_mesh,
    compiler_params=_sc_params,
    scratch_types=[
        pltpu.VMEM_SHARED((NPC, L), f32),
        pltpu.VMEM((CS,), jnp.int32),
        pltpu.VMEM((CS, L), f32),
    ],
)


def _edge_mlp_body(a_hbm, b_hbm, src_hbm, dst_hbm, w2_hbm, out_hbm,
                   sidx, didx, arows, brows, w2v, obuf,
                   sem_a0, sem_a1, sem_b0, sem_b1):
    c = lax.axis_index("c")
    s = lax.axis_index("s")
    w = s * NC + c
    sems_a = (sem_a0, sem_a1)
    sems_b = (sem_b0, sem_b1)

    pltpu.sync_copy(w2_hbm, w2v)
    wv = [w2v[pl.ds(k * L, L)] for k in range(HID // L)]
    b2vec = w2v[pl.ds(HID, L)]
    lane = lax.broadcasted_iota(jnp.int32, (L,), 0)

    nch = jnp.where(w < (NCHE % NW), NCHE // NW + 1, NCHE // NW)

    def fire(k, b):
        cb = (w + k * NW) * CE
        pltpu.sync_copy(src_hbm.at[pl.ds(cb, CE)], sidx.at[b])
        pltpu.sync_copy(dst_hbm.at[pl.ds(cb, CE)], didx.at[b])
        pltpu.async_copy(a_hbm.at[sidx.at[b]], arows.at[b], sems_a[b])
        pltpu.async_copy(b_hbm.at[didx.at[b]], brows.at[b], sems_b[b])

    def drain(k, b):
        cb = (w + k * NW) * CE
        pltpu.make_async_copy(a_hbm.at[sidx.at[b]], arows.at[b], sems_a[b]).wait()
        pltpu.make_async_copy(b_hbm.at[didx.at[b]], brows.at[b], sems_b[b]).wait()

        def group(g, carry2):
            ovec = jnp.zeros((L,), f32)
            for e16 in range(L):
                e = g * L + e16
                acc = b2vec
                for kk in range(HID // L):
                    v = arows[b, e, pl.ds(kk * L, L)] + brows[b, e, pl.ds(kk * L, L)]
                    acc = acc + jnp.maximum(v, 0.0) * wv[kk]
                ovec = jnp.where(lane == e16, jnp.sum(acc), ovec)
            obuf[pl.ds(g * L, L)] = ovec
            return carry2

        lax.fori_loop(0, CE // L, group, 0)
        pltpu.sync_copy(obuf, out_hbm.at[pl.ds(cb, CE)])

    fire(0, 0)
    n_t2 = (NCHE // NW + 2) // 2

    def step(t2, carry):
        for b in (0, 1):
            k = t2 * 2 + b
            pl.when(k + 1 < nch)(lambda: fire(k + 1, 1 - b))
            pl.when(k < nch)(lambda: drain(k, b))
        return carry

    lax.fori_loop(0, n_t2, step, 0)


_edge_mlp_call = pl.kernel(
    _edge_mlp_body,
    out_type=jax.ShapeDtypeStruct((E,), f32),
    mesh=_mesh,
    compiler_params=_sc_params,
    scratch_types=[
        pltpu.VMEM((2, CE), jnp.int32),
        pltpu.VMEM((2, CE), jnp.int32),
        pltpu.VMEM((2, CE, HID), f32),
        pltpu.VMEM((2, CE, HID), f32),
        pltpu.VMEM((80,), f32),
        pltpu.VMEM((CE,), f32),
        pltpu.SemaphoreType.DMA,
        pltpu.SemaphoreType.DMA,
        pltpu.SemaphoreType.DMA,
        pltpu.SemaphoreType.DMA,
    ],
)


R_ENC = 2000


def _dot(a, b):
    return jnp.dot(a, b, precision=jax.lax.Precision.HIGHEST,
                   preferred_element_type=f32)


def _encode_body(x_ref, tg_ref, gg_ref, cg_ref, tr_ref, gr_ref, cr_ref,
                 g1_ref, r1_ref):
    xb = x_ref[...]
    cls = xb[:, 0:1].astype(jnp.int32)
    onehot = (lax.broadcasted_iota(jnp.int32, (R_ENC, NCLS), 1) == cls)
    onehot = onehot.astype(f32)
    geom = xb[:, 1:5]
    g1_ref[...] = _dot(onehot, tg_ref[...]) + _dot(geom, gg_ref[...]) + cg_ref[...]
    r1_ref[...] = _dot(onehot, tr_ref[...]) + _dot(geom, gr_ref[...]) + cr_ref[...]


def _encode(x, tg, gg, cg, tr, gr, cr):
    full = lambda shape: pl.BlockSpec(shape, lambda i: (0, 0))
    return pl.pallas_call(
        _encode_body,
        grid=(N // R_ENC,),
        in_specs=[
            pl.BlockSpec((R_ENC, 5), lambda i: (i, 0)),
            full((NCLS, HID)), full((4, HID)), full((1, HID)),
            full((NCLS, HID)), full((4, HID)), full((1, HID)),
        ],
        out_specs=[
            pl.BlockSpec((R_ENC, HID), lambda i: (i, 0)),
            pl.BlockSpec((R_ENC, HID), lambda i: (i, 0)),
        ],
        out_shape=[
            jax.ShapeDtypeStruct((N, HID), f32),
            jax.ShapeDtypeStruct((N, HID), f32),
        ],
    )(x, tg, gg, cg, tr, gr, cr)


R_FIN = 5000
_HB = HALF // R_FIN


def _recip(c):
    r = 1.0 / c
    return r * (2.0 - c * r)


def _finish1_body(acc_ref, cnt_ref, r1_ref, wl_ref, wr_ref, b_ref,
                  g2_ref, r2_ref):
    cnt = (cnt_ref[0] + cnt_ref[1])[:, 0:1]
    inv = _recip(jnp.maximum(cnt, 1.0))
    h1 = jnp.maximum(acc_ref[0] * inv + r1_ref[...], 0.0)
    g2_ref[...] = _dot(h1, wl_ref[...])
    r2_ref[...] = _dot(h1, wr_ref[...]) + b_ref[...]


def _finish2_body(acc_ref, cnt_ref, r2_ref, wa_ref, wb_ref, b_ref,
                  a_ref, b_out_ref):
    cnt = (cnt_ref[0] + cnt_ref[1])[:, 0:1]
    inv = _recip(jnp.maximum(cnt, 1.0))
    h2 = acc_ref[0] * inv + r2_ref[...]
    a_ref[...] = _dot(h2, wa_ref[...]) + b_ref[...]
    b_out_ref[...] = _dot(h2, wb_ref[...])


def _finish(body, acc, cnt, r, wl, wr, b):
    full = lambda shape: pl.BlockSpec(shape, lambda h, i: (0, 0))
    return pl.pallas_call(
        body,
        grid=(NC, _HB),
        in_specs=[
            pl.BlockSpec((1, R_FIN, HID), lambda h, i: (h, i, 0)),
            pl.BlockSpec((NC, R_FIN, L), lambda h, i: (0, h * _HB + i, 0)),
            pl.BlockSpec((R_FIN, HID), lambda h, i: (h * _HB + i, 0)),
            full((HID, HID)), full((HID, HID)), full((1, HID)),
        ],
        out_specs=[
            pl.BlockSpec((R_FIN, HID), lambda h, i: (h * _HB + i, 0)),
            pl.BlockSpec((R_FIN, HID), lambda h, i: (h * _HB + i, 0)),
        ],
        out_shape=[
            jax.ShapeDtypeStruct((N, HID), f32),
            jax.ShapeDtypeStruct((N, HID), f32),
        ],
    )(acc, cnt, r, wl, wr, b)


def kernel(x, edge_index, class_emb, geom_W, geom_b, W1_l, W1_r, b1,
           W2_l, W2_r, b2, mlp_W1, mlp_b1, mlp_W2, mlp_b2):
    src = edge_index[0]
    dst = edge_index[1]

    hp = jax.lax.Precision.HIGHEST
    mm = lambda a, b: jnp.dot(a, b, precision=hp)
    W1l_a, W1l_b = W1_l[:HID], W1_l[HID:]
    W1r_a, W1r_b = W1_r[:HID], W1_r[HID:]
    tg = mm(class_emb, W1l_a)
    gg = mm(geom_W, W1l_b)
    cg = mm(geom_b[None], W1l_b)
    tr = mm(class_emb, W1r_a)
    gr = mm(geom_W, W1r_b)
    cr = mm(geom_b[None], W1r_b) + b1[None]
    wa = mlp_W1[:HID]
    wb = mlp_W1[HID:]
    w2pad = jnp.zeros((80,), f32).at[:HID].set(mlp_W2[:, 0]).at[HID].set(mlp_b2[0])

    z64 = jnp.zeros((STRIPE, HID), f32)
    z16 = jnp.zeros((SP_C, L), f32)
    ones = jnp.ones((CS, L), f32)

    g1, r1 = _encode(x, tg, gg, cg, tr, gr, cr)
    psrc, pdst, meta = _partition_call(src, dst)
    cnt = _count_call(dst, z16, ones)
    acc1 = _segsum_call(g1, psrc, pdst, meta, z64)
    g2, r2 = _finish(_finish1_body, acc1, cnt, r1, W2_l, W2_r, b2[None])
    acc2 = _segsum_call(g2, psrc, pdst, meta, z64)
    a_tab, b_tab = _finish(_finish2_body, acc2, cnt, r2, wa, wb, mlp_b1[None])
    return _edge_mlp_call(a_tab, b_tab, src, dst, w2pad)


# speedup vs baseline: 9.8611x; 1.0482x over previous
import functools

import jax
import jax.numpy as jnp
from jax import lax
from jax.experimental import pallas as pl
from jax.experimental.pallas import tpu as pltpu
from jax.experimental.pallas import tpu_sc as plsc

N = 50000
E = 800000
HID = 64
NCLS = 32

NC = 2
NS = 16
L = 16

HALF = N // NC
DUMMY = HALF
STRIPE = 1568
HP = NS * STRIPE
SP_C = 3128
NPC = NS * SP_C
CS = 160
NCHS = E // CS
CE = 320
NCHE = E // CE
NW = NC * NS
EPT = E // NS
CP = 1000
ECAP = E + NS * CS
ST16 = HP // NS

_mesh = plsc.VectorSubcoreMesh(
    core_axis_name="c", subcore_axis_name="s", num_cores=NC, num_subcores=NS)

_sc_params = pltpu.CompilerParams(use_tc_tiling_on_sc=False,
                                  needs_layout_passes=False)

f32 = jnp.float32


def _partition_body(src_hbm, dst_hbm, z16_hbm, ones_hbm,
                    psrc_out, pdst_out, meta_out, cnt_out,
                    ssrc, sdst, sbuf, dbuf, mbuf, cnt_sh, ones_v, cntr):
    c = lax.axis_index("c")
    s = lax.axis_index("s")
    lo = c * HALF
    lane = lax.broadcasted_iota(jnp.int32, (L,), 0)

    @pl.when(s == 0)
    def _():
        cntr[0] = 0
    pltpu.sync_copy(z16_hbm, cnt_sh.at[pl.ds(s * ST16, ST16)])
    pltpu.sync_copy(ones_hbm, ones_v)
    plsc.subcore_barrier()

    base_e = s * EPT

    def scan_chunk(q, n):
        pltpu.sync_copy(src_hbm.at[pl.ds(base_e + q * CP, CP)], sbuf)
        pltpu.sync_copy(dst_hbm.at[pl.ds(base_e + q * CP, CP)], dbuf)

        def group(j, n2):
            sv = sbuf[pl.ds(j * L, L)]
            dv = dbuf[pl.ds(j * L, L)]
            dloc = dv - lo
            m = (dloc >= 0) & (dloc < HALF)
            plsc.store_compressed(ssrc.at[pl.ds(n2, L)], sv, mask=m)
            plsc.store_compressed(sdst.at[pl.ds(n2, L)], dloc, mask=m)
            return n2 + jnp.sum(m.astype(jnp.int32))

        return lax.fori_loop(0, CP // L, group, n)

    n_loc = lax.fori_loop(0, EPT // CP, scan_chunk, jnp.int32(0))

    zl = jnp.zeros((L,), jnp.int32)
    dl = jnp.full((L,), DUMMY, jnp.int32)
    for j in range(CS // L):
        ssrc[pl.ds(n_loc + j * L, L)] = zl
        sdst[pl.ds(n_loc + j * L, L)] = dl
    nch_loc = (n_loc + CS - 1) // CS
    base = plsc.fetch_and_add(cntr.at[0], nch_loc * CS, subcore_id=0)

    def flush(i, carry):
        ob = pl.multiple_of(base + i * CS, CS)
        pltpu.sync_copy(ssrc.at[pl.ds(i * CS, CS)],
                        psrc_out.at[c, pl.ds(ob, CS)])
        pltpu.sync_copy(sdst.at[pl.ds(i * CS, CS)],
                        pdst_out.at[c, pl.ds(ob, CS)])
        pltpu.sync_copy(ones_v, cnt_sh.at[sdst.at[pl.ds(i * CS, CS)]], add=True)
        return carry

    lax.fori_loop(0, nch_loc, flush, 0)
    plsc.subcore_barrier()

    pltpu.sync_copy(cnt_sh.at[pl.ds(s * ST16, ST16)],
                    cnt_out.at[c, pl.ds(s * ST16, ST16)])

    @pl.when(s == 0)
    def _():
        total = plsc.fetch_and_add(cntr.at[0], 0, subcore_id=0)
        mbuf[...] = jnp.where(lane == 0, total, 0)
        pltpu.sync_copy(mbuf, meta_out.at[c])


_partition_call = pl.kernel(
    _partition_body,
    out_type=(jax.ShapeDtypeStruct((NC, ECAP), jnp.int32),
              jax.ShapeDtypeStruct((NC, ECAP), jnp.int32),
              jax.ShapeDtypeStruct((NC, L), jnp.int32),
              jax.ShapeDtypeStruct((NC, HP, L), f32)),
    mesh=_mesh,
    compiler_params=_sc_params,
    scratch_types=[
        pltpu.VMEM((EPT + CS,), jnp.int32),
        pltpu.VMEM((EPT + CS,), jnp.int32),
        pltpu.VMEM((CP,), jnp.int32),
        pltpu.VMEM((CP,), jnp.int32),
        pltpu.VMEM((L,), jnp.int32),
        pltpu.VMEM_SHARED((HP, L), f32),
        pltpu.VMEM((CS, L), f32),
        pltpu.SMEM((1,), jnp.int32),
    ],
)


def _segsum_body(g_hbm, psrc_hbm, pdst_hbm, meta_hbm, z64_hbm,
                 acc_out,
                 acc_sh, sidx, didx, rows, mbuf, sem0, sem1):
    c = lax.axis_index("c")
    s = lax.axis_index("s")
    base_r = s * STRIPE
    sems = (sem0, sem1)

    pltpu.sync_copy(z64_hbm, acc_sh.at[pl.ds(base_r, STRIPE)])
    plsc.subcore_barrier()

    pltpu.sync_copy(meta_hbm.at[c], mbuf)
    total = mbuf[...][0]
    nchunks = total // CS
    nch = (nchunks - s + NS - 1) // NS

    def fire(k, b):
        cb = pl.multiple_of((s + k * NS) * CS, CS)
        pltpu.sync_copy(psrc_hbm.at[c, pl.ds(cb, CS)], sidx.at[b])
        pltpu.sync_copy(pdst_hbm.at[c, pl.ds(cb, CS)], didx.at[b])
        pltpu.async_copy(g_hbm.at[sidx.at[b]], rows.at[b], sems[b])

    def drain(b):
        pltpu.make_async_copy(g_hbm.at[sidx.at[b]], rows.at[b], sems[b]).wait()
        pltpu.sync_copy(rows.at[b], acc_sh.at[didx.at[b]], add=True)

    pl.when(nch > 0)(lambda: fire(0, 0))
    n_t2 = 158

    def step(t2, carry):
        for b in (0, 1):
            k = t2 * 2 + b
            pl.when(k + 1 < nch)(lambda: fire(k + 1, 1 - b))
            pl.when(k < nch)(lambda: drain(b))
        return carry

    lax.fori_loop(0, n_t2, step, 0)
    plsc.subcore_barrier()

    pltpu.sync_copy(acc_sh.at[pl.ds(base_r, STRIPE)],
                    acc_out.at[c, pl.ds(base_r, STRIPE)])


_segsum_call = pl.kernel(
    _segsum_body,
    out_type=jax.ShapeDtypeStruct((NC, HP, HID), f32),
    mesh=_mesh,
    compiler_params=_sc_params,
    scratch_types=[
        pltpu.VMEM_SHARED((HP, HID), f32),
        pltpu.VMEM((2, CS), jnp.int32),
        pltpu.VMEM((2, CS), jnp.int32),
        pltpu.VMEM((2, CS, HID), f32),
        pltpu.VMEM((L,), jnp.int32),
        pltpu.SemaphoreType.DMA,
        pltpu.SemaphoreType.DMA,
    ],
)


def _edge_mlp_body(a_hbm, b_hbm, src_hbm, dst_hbm, w2_hbm, out_hbm,
                   sidx, didx, arows, brows, w2v, obuf,
                   sem_a0, sem_a1, sem_b0, sem_b1):
    c = lax.axis_index("c")
    s = lax.axis_index("s")
    w = s * NC + c
    sems_a = (sem_a0, sem_a1)
    sems_b = (sem_b0, sem_b1)

    pltpu.sync_copy(w2_hbm, w2v)
    wv = [w2v[pl.ds(k * L, L)] for k in range(HID // L)]
    b2vec = w2v[pl.ds(HID, L)]
    lane = lax.broadcasted_iota(jnp.int32, (L,), 0)

    nch = jnp.where(w < (NCHE % NW), NCHE // NW + 1, NCHE // NW)

    def fire(k, b):
        cb = (w + k * NW) * CE
        pltpu.sync_copy(src_hbm.at[pl.ds(cb, CE)], sidx.at[b])
        pltpu.sync_copy(dst_hbm.at[pl.ds(cb, CE)], didx.at[b])
        pltpu.async_copy(a_hbm.at[sidx.at[b]], arows.at[b], sems_a[b])
        pltpu.async_copy(b_hbm.at[didx.at[b]], brows.at[b], sems_b[b])

    def drain(k, b):
        cb = (w + k * NW) * CE
        pltpu.make_async_copy(a_hbm.at[sidx.at[b]], arows.at[b], sems_a[b]).wait()
        pltpu.make_async_copy(b_hbm.at[didx.at[b]], brows.at[b], sems_b[b]).wait()

        def group(g, carry2):
            ovec = jnp.zeros((L,), f32)
            for e16 in range(L):
                e = g * L + e16
                acc = b2vec
                for kk in range(HID // L):
                    v = arows[b, e, pl.ds(kk * L, L)] + brows[b, e, pl.ds(kk * L, L)]
                    acc = acc + jnp.maximum(v, 0.0) * wv[kk]
                ovec = jnp.where(lane == e16, jnp.sum(acc), ovec)
            obuf[pl.ds(g * L, L)] = ovec
            return carry2

        lax.fori_loop(0, CE // L, group, 0)
        pltpu.sync_copy(obuf, out_hbm.at[pl.ds(cb, CE)])

    fire(0, 0)
    n_t2 = (NCHE // NW + 2) // 2

    def step(t2, carry):
        for b in (0, 1):
            k = t2 * 2 + b
            pl.when(k + 1 < nch)(lambda: fire(k + 1, 1 - b))
            pl.when(k < nch)(lambda: drain(k, b))
        return carry

    lax.fori_loop(0, n_t2, step, 0)


_edge_mlp_call = pl.kernel(
    _edge_mlp_body,
    out_type=jax.ShapeDtypeStruct((E,), f32),
    mesh=_mesh,
    compiler_params=_sc_params,
    scratch_types=[
        pltpu.VMEM((2, CE), jnp.int32),
        pltpu.VMEM((2, CE), jnp.int32),
        pltpu.VMEM((2, CE, HID), f32),
        pltpu.VMEM((2, CE, HID), f32),
        pltpu.VMEM((80,), f32),
        pltpu.VMEM((CE,), f32),
        pltpu.SemaphoreType.DMA,
        pltpu.SemaphoreType.DMA,
        pltpu.SemaphoreType.DMA,
        pltpu.SemaphoreType.DMA,
    ],
)


R_ENC = 2000


def _dot(a, b):
    return jnp.dot(a, b, precision=jax.lax.Precision.HIGHEST,
                   preferred_element_type=f32)


def _encode_body(x_ref, tg_ref, gg_ref, cg_ref, tr_ref, gr_ref, cr_ref,
                 g1_ref, r1_ref):
    xb = x_ref[...]
    cls = xb[:, 0:1].astype(jnp.int32)
    onehot = (lax.broadcasted_iota(jnp.int32, (R_ENC, NCLS), 1) == cls)
    onehot = onehot.astype(f32)
    geom = xb[:, 1:5]
    g1_ref[...] = _dot(onehot, tg_ref[...]) + _dot(geom, gg_ref[...]) + cg_ref[...]
    r1_ref[...] = _dot(onehot, tr_ref[...]) + _dot(geom, gr_ref[...]) + cr_ref[...]


def _encode(x, tg, gg, cg, tr, gr, cr):
    full = lambda shape: pl.BlockSpec(shape, lambda i: (0, 0))
    return pl.pallas_call(
        _encode_body,
        grid=(N // R_ENC,),
        in_specs=[
            pl.BlockSpec((R_ENC, 5), lambda i: (i, 0)),
            full((NCLS, HID)), full((4, HID)), full((1, HID)),
            full((NCLS, HID)), full((4, HID)), full((1, HID)),
        ],
        out_specs=[
            pl.BlockSpec((R_ENC, HID), lambda i: (i, 0)),
            pl.BlockSpec((R_ENC, HID), lambda i: (i, 0)),
        ],
        out_shape=[
            jax.ShapeDtypeStruct((N, HID), f32),
            jax.ShapeDtypeStruct((N, HID), f32),
        ],
    )(x, tg, gg, cg, tr, gr, cr)


R_FIN = 5000
_HB = HALF // R_FIN


def _recip(c):
    r = 1.0 / c
    return r * (2.0 - c * r)


def _finish1_body(acc_ref, cnt_ref, r1_ref, wl_ref, wr_ref, b_ref,
                  g2_ref, r2_ref):
    cnt = cnt_ref[0][:, 0:1]
    inv = _recip(jnp.maximum(cnt, 1.0))
    h1 = jnp.maximum(acc_ref[0] * inv + r1_ref[...], 0.0)
    g2_ref[...] = _dot(h1, wl_ref[...])
    r2_ref[...] = _dot(h1, wr_ref[...]) + b_ref[...]


def _finish2_body(acc_ref, cnt_ref, r2_ref, wa_ref, wb_ref, b_ref,
                  a_ref, b_out_ref):
    cnt = cnt_ref[0][:, 0:1]
    inv = _recip(jnp.maximum(cnt, 1.0))
    h2 = acc_ref[0] * inv + r2_ref[...]
    a_ref[...] = _dot(h2, wa_ref[...]) + b_ref[...]
    b_out_ref[...] = _dot(h2, wb_ref[...])


def _finish(body, acc, cnt, r, wl, wr, b):
    full = lambda shape: pl.BlockSpec(shape, lambda h, i: (0, 0))
    return pl.pallas_call(
        body,
        grid=(NC, _HB),
        in_specs=[
            pl.BlockSpec((1, R_FIN, HID), lambda h, i: (h, i, 0)),
            pl.BlockSpec((1, R_FIN, L), lambda h, i: (h, i, 0)),
            pl.BlockSpec((R_FIN, HID), lambda h, i: (h * _HB + i, 0)),
            full((HID, HID)), full((HID, HID)), full((1, HID)),
        ],
        out_specs=[
            pl.BlockSpec((R_FIN, HID), lambda h, i: (h * _HB + i, 0)),
            pl.BlockSpec((R_FIN, HID), lambda h, i: (h * _HB + i, 0)),
        ],
        out_shape=[
            jax.ShapeDtypeStruct((N, HID), f32),
            jax.ShapeDtypeStruct((N, HID), f32),
        ],
    )(acc, cnt, r, wl, wr, b)


def kernel(x, edge_index, class_emb, geom_W, geom_b, W1_l, W1_r, b1,
           W2_l, W2_r, b2, mlp_W1, mlp_b1, mlp_W2, mlp_b2):
    src = edge_index[0]
    dst = edge_index[1]

    hp = jax.lax.Precision.HIGHEST
    mm = lambda a, b: jnp.dot(a, b, precision=hp)
    W1l_a, W1l_b = W1_l[:HID], W1_l[HID:]
    W1r_a, W1r_b = W1_r[:HID], W1_r[HID:]
    tg = mm(class_emb, W1l_a)
    gg = mm(geom_W, W1l_b)
    cg = mm(geom_b[None], W1l_b)
    tr = mm(class_emb, W1r_a)
    gr = mm(geom_W, W1r_b)
    cr = mm(geom_b[None], W1r_b) + b1[None]
    wa = mlp_W1[:HID]
    wb = mlp_W1[HID:]
    w2pad = jnp.zeros((80,), f32).at[:HID].set(mlp_W2[:, 0]).at[HID].set(mlp_b2[0])

    z64 = jnp.zeros((STRIPE, HID), f32)
    z16 = jnp.zeros((ST16, L), f32)
    ones = jnp.ones((CS, L), f32)

    g1, r1 = _encode(x, tg, gg, cg, tr, gr, cr)
    psrc, pdst, meta, cnt = _partition_call(src, dst, z16, ones)
    acc1 = _segsum_call(g1, psrc, pdst, meta, z64)
    g2, r2 = _finish(_finish1_body, acc1, cnt, r1, W2_l, W2_r, b2[None])
    acc2 = _segsum_call(g2, psrc, pdst, meta, z64)
    a_tab, b_tab = _finish(_finish2_body, acc2, cnt, r2, wa, wb, mlp_b1[None])
    return _edge_mlp_call(a_tab, b_tab, src, dst, w2pad)
